# Initial kernel scaffold; baseline (speedup 1.0000x reference)
import jax
import jax.numpy as jnp
from jax.experimental import pallas as pl


def kernel(x, gfeat, edge_index, W_self0, W_neigh0, b0, W_self1, W_neigh1, b1, proto):
    raise NotImplementedError("write your pallas kernel here")


# speedup vs baseline: 4.6927x; 4.6927x over previous
import functools

import jax
import jax.numpy as jnp
from jax import lax
from jax.experimental import pallas as pl
from jax.experimental.pallas import tpu as pltpu
from jax.experimental.pallas import tpu_sc as plsc

N = 10000
E = 320000
H = 128
C = 16

NC, NS = 2, 16
NW = NC * NS
EW = E // NW
K = 80
NCHUNK = EW // K
NP = 10240
RPW = NP // NS
DEGW = 128


def _make_sc_agg(mesh):
    out_type = jax.ShapeDtypeStruct((NC * NP, H), jnp.float32)
    scratch = [
        pltpu.VMEM((K,), jnp.int32),
        pltpu.VMEM((K,), jnp.int32),
        pltpu.VMEM((K, H), jnp.float32),
        pltpu.VMEM_SHARED((NP, H), jnp.float32),
        pltpu.SemaphoreType.DMA,
    ]

    def body(h_hbm, src_hbm, dst_hbm, z_hbm,
             out_hbm, src_v, dst_v, rows_v, agg_sh, sem):
        c = lax.axis_index("c")
        s = lax.axis_index("s")
        wid = s * NC + c

        row0 = pl.multiple_of(s * RPW, 8)
        pltpu.sync_copy(z_hbm, agg_sh.at[pl.ds(row0, RPW)])
        plsc.subcore_barrier()

        def step(i, carry):
            base = pl.multiple_of(wid * EW + i * K, 8)
            pltpu.sync_copy(src_hbm.at[pl.ds(base, K)], src_v)
            pltpu.sync_copy(dst_hbm.at[pl.ds(base, K)], dst_v)
            pltpu.async_copy(h_hbm.at[src_v], rows_v, sem).wait()
            pltpu.sync_copy(rows_v, agg_sh.at[dst_v], add=True)
            return carry

        lax.fori_loop(0, NCHUNK, step, 0)
        plsc.subcore_barrier()

        pltpu.sync_copy(agg_sh.at[pl.ds(row0, RPW)],
                        out_hbm.at[pl.ds(c * NP + row0, RPW)])

    return pl.kernel(body, out_type=out_type, mesh=mesh,
                     scratch_types=scratch)


def _make_sc_deg(mesh):
    out_type = jax.ShapeDtypeStruct((NC * NP, DEGW), jnp.float32)
    scratch = [
        pltpu.VMEM((K,), jnp.int32),
        pltpu.VMEM((K, DEGW), jnp.float32),
        pltpu.VMEM_SHARED((NP, DEGW), jnp.float32),
        pltpu.SemaphoreType.DMA,
    ]

    def body(dst_hbm, z_hbm, ones_hbm,
             out_hbm, dst_v, ones_v, deg_sh, sem):
        c = lax.axis_index("c")
        s = lax.axis_index("s")
        wid = s * NC + c

        row0 = pl.multiple_of(s * RPW, 8)
        pltpu.sync_copy(z_hbm, deg_sh.at[pl.ds(row0, RPW)])
        pltpu.sync_copy(ones_hbm, ones_v)
        plsc.subcore_barrier()

        def step(i, carry):
            base = pl.multiple_of(wid * EW + i * K, 8)
            pltpu.sync_copy(dst_hbm.at[pl.ds(base, K)], dst_v)
            pltpu.sync_copy(ones_v, deg_sh.at[dst_v], add=True)
            return carry

        lax.fori_loop(0, NCHUNK, step, 0)
        plsc.subcore_barrier()

        pltpu.sync_copy(deg_sh.at[pl.ds(row0, RPW)],
                        out_hbm.at[pl.ds(c * NP + row0, RPW)])

    return pl.kernel(body, out_type=out_type, mesh=mesh,
                     scratch_types=scratch)


@functools.cache
def _get_sc_kernels():
    mesh = plsc.VectorSubcoreMesh(core_axis_name="c", subcore_axis_name="s",
                                  num_cores=NC, num_subcores=NS)
    return _make_sc_agg(mesh), _make_sc_deg(mesh)


_R = 1000


def _dense_body(h_ref, p_ref, deg_ref, ws_ref, wn_ref, b_ref, o_ref):
    p = p_ref[...]
    agg = p[0] + p[1]
    d = deg_ref[...]
    deg = jnp.maximum((d[0] + d[1])[:, 0:1], 1.0)
    hn = agg / deg
    acc = jnp.dot(h_ref[...], ws_ref[...], preferred_element_type=jnp.float32)
    acc = acc + jnp.dot(hn, wn_ref[...], preferred_element_type=jnp.float32)
    o_ref[...] = jnp.maximum(acc + b_ref[...], 0.0)


_dense = pl.pallas_call(
    _dense_body,
    grid=(N // _R,),
    in_specs=[
        pl.BlockSpec((_R, H), lambda i: (i, 0)),
        pl.BlockSpec((NC, _R, H), lambda i: (0, i, 0)),
        pl.BlockSpec((NC, _R, DEGW), lambda i: (0, i, 0)),
        pl.BlockSpec((H, H), lambda i: (0, 0)),
        pl.BlockSpec((H, H), lambda i: (0, 0)),
        pl.BlockSpec((1, H), lambda i: (0, 0)),
    ],
    out_specs=pl.BlockSpec((_R, H), lambda i: (i, 0)),
    out_shape=jax.ShapeDtypeStruct((N, H), jnp.float32),
)


def _head_body(h_ref, p_ref, deg_ref, ws_ref, wn_ref, b_ref, g_ref,
               pht_ref, pgt_ref, o_ref):
    p = p_ref[...]
    agg = p[0] + p[1]
    d = deg_ref[...]
    deg = jnp.maximum((d[0] + d[1])[:, 0:1], 1.0)
    hn = agg / deg
    acc = jnp.dot(h_ref[...], ws_ref[...], preferred_element_type=jnp.float32)
    acc = acc + jnp.dot(hn, wn_ref[...], preferred_element_type=jnp.float32)
    h2 = jnp.maximum(acc + b_ref[...], 0.0)
    g = g_ref[...]
    pht = pht_ref[...]
    pgt = pgt_ref[...]
    psq = jnp.sum(pht * pht, axis=0) + jnp.sum(pgt * pgt, axis=0)
    sq = (jnp.sum(h2 * h2, axis=1, keepdims=True)
          + jnp.sum(g * g, axis=1, keepdims=True))
    dotp = (jnp.dot(h2, pht, preferred_element_type=jnp.float32)
            + jnp.dot(g, pgt, preferred_element_type=jnp.float32))
    o_ref[...] = 2.0 * dotp - sq - psq[None, :]


_head = pl.pallas_call(
    _head_body,
    grid=(N // _R,),
    in_specs=[
        pl.BlockSpec((_R, H), lambda i: (i, 0)),
        pl.BlockSpec((NC, _R, H), lambda i: (0, i, 0)),
        pl.BlockSpec((NC, _R, DEGW), lambda i: (0, i, 0)),
        pl.BlockSpec((H, H), lambda i: (0, 0)),
        pl.BlockSpec((H, H), lambda i: (0, 0)),
        pl.BlockSpec((1, H), lambda i: (0, 0)),
        pl.BlockSpec((_R, H), lambda i: (i, 0)),
        pl.BlockSpec((H, C), lambda i: (0, 0)),
        pl.BlockSpec((H, C), lambda i: (0, 0)),
    ],
    out_specs=pl.BlockSpec((_R, C), lambda i: (i, 0)),
    out_shape=jax.ShapeDtypeStruct((N, C), jnp.float32),
)


def kernel(x, gfeat, edge_index, W_self0, W_neigh0, b0,
           W_self1, W_neigh1, b1, proto):
    src = edge_index[0]
    dst = edge_index[1]
    z_h = jnp.zeros((RPW, H), jnp.float32)
    z_d = jnp.zeros((RPW, DEGW), jnp.float32)
    ones = jnp.ones((K, DEGW), jnp.float32)
    b0_2 = b0.reshape(1, H)
    b1_2 = b1.reshape(1, H)
    pht = proto[:, :H].T
    pgt = proto[:, H:].T

    _sc_agg, _sc_deg = _get_sc_kernels()
    degp = _sc_deg(dst, z_d, ones).reshape(NC, NP, DEGW)
    p0 = _sc_agg(x, src, dst, z_h).reshape(NC, NP, H)
    h1 = _dense(x, p0, degp, W_self0, W_neigh0, b0_2)
    p1 = _sc_agg(h1, src, dst, z_h).reshape(NC, NP, H)
    logits = _head(h1, p1, degp, W_self1, W_neigh1, b1_2, gfeat, pht, pgt)
    return logits


# speedup vs baseline: 10.5566x; 2.2496x over previous
import functools

import jax
import jax.numpy as jnp
from jax import lax
from jax.experimental import pallas as pl
from jax.experimental.pallas import tpu as pltpu
from jax.experimental.pallas import tpu_sc as plsc

N = 10000
E = 320000
H = 128
C = 16

NC, NS = 2, 16
NW = NC * NS
EW = E // NW
KC = 125
EC = E // KC
CW = EC // NW
GW = CW // 8
NP = 10240
RPW = NP // NS
DEGW = 128
DGRP = 8


def _make_sc_agg(mesh):
    out_type = jax.ShapeDtypeStruct((NC * NP, H), jnp.float32)
    scratch = [
        pltpu.VMEM((8, KC), jnp.int32),
        pltpu.VMEM((8, KC), jnp.int32),
        pltpu.VMEM((CW, KC), jnp.int32),
        pltpu.VMEM((KC, H), jnp.float32),
        pltpu.VMEM((KC, H), jnp.float32),
        pltpu.VMEM_SHARED((NP, H), jnp.float32),
        pltpu.SemaphoreType.DMA,
        pltpu.SemaphoreType.DMA,
        pltpu.SemaphoreType.DMA,
    ]

    def body(h_hbm, src_hbm, dst_hbm, z_hbm, out_hbm,
             srcg0, srcg1, dst_v, rows0, rows1, agg_sh, sem0, sem1, semg):
        c = lax.axis_index("c")
        s = lax.axis_index("s")
        wid = s * NC + c
        crow = pl.multiple_of(wid * CW, 8)

        row0 = pl.multiple_of(s * RPW, 8)
        pltpu.sync_copy(z_hbm, agg_sh.at[pl.ds(row0, RPW)])
        pltpu.sync_copy(dst_hbm.at[pl.ds(crow, CW)], dst_v)
        pltpu.sync_copy(src_hbm.at[pl.ds(crow, 8)], srcg0)
        pltpu.async_copy(src_hbm.at[pl.ds(crow + 8, 8)], srcg1, semg)
        plsc.subcore_barrier()

        bufs = ((rows0, sem0), (rows1, sem1))
        srcgs = (srcg0, srcg1)
        for b, (rows, sem) in enumerate(bufs):
            pltpu.async_copy(h_hbm.at[srcg0.at[b]], rows, sem)

        def g_step(g, carry):
            for j in range(8):
                i = g * 8 + j
                rows, sem = bufs[j % 2]
                pltpu.make_async_copy(h_hbm.at[srcg0.at[j]], rows, sem).wait()
                pltpu.sync_copy(rows, agg_sh.at[dst_v.at[i]], add=True)

                if j < 6:
                    for b in range(2):
                        @pl.when(lax.rem(g, 2) == b)
                        def _():
                            pltpu.async_copy(h_hbm.at[srcgs[b].at[j + 2]],
                                             rows, sem)
                else:
                    if j == 6:
                        @pl.when(g + 1 < GW)
                        def _():
                            pltpu.make_async_copy(
                                src_hbm.at[pl.ds(crow, 8)], srcg1, semg).wait()

                    @pl.when(g + 1 < GW)
                    def _():
                        for b in range(2):
                            @pl.when(lax.rem(g + 1, 2) == b)
                            def _():
                                pltpu.async_copy(h_hbm.at[srcgs[b].at[j - 6]],
                                                 rows, sem)

            @pl.when(g + 2 < GW)
            def _():
                grow = pl.multiple_of(crow + (g + 2) * 8, 8)
                for b in range(2):
                    @pl.when(lax.rem(g, 2) == b)
                    def _():
                        pltpu.async_copy(src_hbm.at[pl.ds(grow, 8)],
                                         srcgs[b], semg)
            return carry

        lax.fori_loop(0, GW, g_step, 0)
        plsc.subcore_barrier()

        pltpu.sync_copy(agg_sh.at[pl.ds(row0, RPW)],
                        out_hbm.at[pl.ds(c * NP + row0, RPW)])

    return pl.kernel(body, out_type=out_type, mesh=mesh,
                     scratch_types=scratch)


def _make_sc_deg(mesh):
    out_type = jax.ShapeDtypeStruct((NC * NP, DEGW), jnp.float32)
    scratch = [
        pltpu.VMEM((CW, KC), jnp.int32),
        pltpu.VMEM((KC, DEGW), jnp.float32),
        pltpu.VMEM_SHARED((NP, DEGW), jnp.float32),
        pltpu.SemaphoreType.DMA,
    ]

    def body(dst_hbm, z_hbm, ones_hbm,
             out_hbm, dst_v, ones_v, deg_sh, sem):
        c = lax.axis_index("c")
        s = lax.axis_index("s")
        wid = s * NC + c

        row0 = pl.multiple_of(s * RPW, 8)
        pltpu.sync_copy(z_hbm, deg_sh.at[pl.ds(row0, RPW)])
        pltpu.sync_copy(dst_hbm.at[pl.ds(wid * CW, CW)], dst_v)
        pltpu.sync_copy(ones_hbm, ones_v)
        plsc.subcore_barrier()

        def step(g, carry):
            for j in range(DGRP):
                pltpu.async_copy(ones_v, deg_sh.at[dst_v.at[g * DGRP + j]],
                                 sem, add=True)
            for j in range(DGRP):
                pltpu.make_async_copy(ones_v, deg_sh.at[dst_v.at[0]],
                                      sem).wait()
            return carry

        lax.fori_loop(0, CW // DGRP, step, 0)
        plsc.subcore_barrier()

        pltpu.sync_copy(deg_sh.at[pl.ds(row0, RPW)],
                        out_hbm.at[pl.ds(c * NP + row0, RPW)])

    return pl.kernel(body, out_type=out_type, mesh=mesh,
                     scratch_types=scratch)


@functools.cache
def _get_sc_kernels():
    mesh = plsc.VectorSubcoreMesh(core_axis_name="c", subcore_axis_name="s",
                                  num_cores=NC, num_subcores=NS)
    return _make_sc_agg(mesh), _make_sc_deg(mesh)


_R = 1000


def _dense_body(h_ref, p_ref, deg_ref, ws_ref, wn_ref, b_ref, o_ref):
    p = p_ref[...]
    agg = p[0] + p[1]
    d = deg_ref[...]
    deg = jnp.maximum((d[0] + d[1])[:, 0:1], 1.0)
    hn = agg / deg
    acc = jnp.dot(h_ref[...], ws_ref[...], preferred_element_type=jnp.float32)
    acc = acc + jnp.dot(hn, wn_ref[...], preferred_element_type=jnp.float32)
    o_ref[...] = jnp.maximum(acc + b_ref[...], 0.0)


_dense = pl.pallas_call(
    _dense_body,
    grid=(N // _R,),
    in_specs=[
        pl.BlockSpec((_R, H), lambda i: (i, 0)),
        pl.BlockSpec((NC, _R, H), lambda i: (0, i, 0)),
        pl.BlockSpec((NC, _R, DEGW), lambda i: (0, i, 0)),
        pl.BlockSpec((H, H), lambda i: (0, 0)),
        pl.BlockSpec((H, H), lambda i: (0, 0)),
        pl.BlockSpec((1, H), lambda i: (0, 0)),
    ],
    out_specs=pl.BlockSpec((_R, H), lambda i: (i, 0)),
    out_shape=jax.ShapeDtypeStruct((N, H), jnp.float32),
)


def _head_body(h_ref, p_ref, deg_ref, ws_ref, wn_ref, b_ref, g_ref,
               pht_ref, pgt_ref, o_ref):
    p = p_ref[...]
    agg = p[0] + p[1]
    d = deg_ref[...]
    deg = jnp.maximum((d[0] + d[1])[:, 0:1], 1.0)
    hn = agg / deg
    acc = jnp.dot(h_ref[...], ws_ref[...], preferred_element_type=jnp.float32)
    acc = acc + jnp.dot(hn, wn_ref[...], preferred_element_type=jnp.float32)
    h2 = jnp.maximum(acc + b_ref[...], 0.0)
    g = g_ref[...]
    pht = pht_ref[...]
    pgt = pgt_ref[...]
    psq = jnp.sum(pht * pht, axis=0) + jnp.sum(pgt * pgt, axis=0)
    sq = (jnp.sum(h2 * h2, axis=1, keepdims=True)
          + jnp.sum(g * g, axis=1, keepdims=True))
    dotp = (jnp.dot(h2, pht, preferred_element_type=jnp.float32)
            + jnp.dot(g, pgt, preferred_element_type=jnp.float32))
    o_ref[...] = 2.0 * dotp - sq - psq[None, :]


_head = pl.pallas_call(
    _head_body,
    grid=(N // _R,),
    in_specs=[
        pl.BlockSpec((_R, H), lambda i: (i, 0)),
        pl.BlockSpec((NC, _R, H), lambda i: (0, i, 0)),
        pl.BlockSpec((NC, _R, DEGW), lambda i: (0, i, 0)),
        pl.BlockSpec((H, H), lambda i: (0, 0)),
        pl.BlockSpec((H, H), lambda i: (0, 0)),
        pl.BlockSpec((1, H), lambda i: (0, 0)),
        pl.BlockSpec((_R, H), lambda i: (i, 0)),
        pl.BlockSpec((H, C), lambda i: (0, 0)),
        pl.BlockSpec((H, C), lambda i: (0, 0)),
    ],
    out_specs=pl.BlockSpec((_R, C), lambda i: (i, 0)),
    out_shape=jax.ShapeDtypeStruct((N, C), jnp.float32),
)


def kernel(x, gfeat, edge_index, W_self0, W_neigh0, b0,
           W_self1, W_neigh1, b1, proto):
    src = edge_index[0].reshape(EC, KC)
    dst = edge_index[1].reshape(EC, KC)
    z_h = jnp.zeros((RPW, H), jnp.float32)
    z_d = jnp.zeros((RPW, DEGW), jnp.float32)
    ones = jnp.ones((KC, DEGW), jnp.float32)
    b0_2 = b0.reshape(1, H)
    b1_2 = b1.reshape(1, H)
    pht = proto[:, :H].T
    pgt = proto[:, H:].T

    _sc_agg, _sc_deg = _get_sc_kernels()
    degp = _sc_deg(dst, z_d, ones).reshape(NC, NP, DEGW)
    p0 = _sc_agg(x, src, dst, z_h).reshape(NC, NP, H)
    h1 = _dense(x, p0, degp, W_self0, W_neigh0, b0_2)
    p1 = _sc_agg(h1, src, dst, z_h).reshape(NC, NP, H)
    logits = _head(h1, p1, degp, W_self1, W_neigh1, b1_2, gfeat, pht, pgt)
    return logits


# speedup vs baseline: 10.5789x; 1.0021x over previous
import functools

import jax
import jax.numpy as jnp
from jax import lax
from jax.experimental import pallas as pl
from jax.experimental.pallas import tpu as pltpu
from jax.experimental.pallas import tpu_sc as plsc

N = 10000
E = 320000
H = 128
C = 16

NC, NS = 2, 16
NW = NC * NS
EW = E // NW
KC = 125
EC = E // KC
CW = EC // NW
GW = CW // 8
NP = 10240
RPW = NP // NS
DEGW = 128
DGRP = 8


def _make_sc_agg(mesh):
    out_type = jax.ShapeDtypeStruct((NC * NP, H), jnp.float32)
    scratch = [
        pltpu.VMEM((8, KC), jnp.int32),
        pltpu.VMEM((8, KC), jnp.int32),
        pltpu.VMEM((CW, KC), jnp.int32),
        pltpu.VMEM((KC, H), jnp.float32),
        pltpu.VMEM((KC, H), jnp.float32),
        pltpu.VMEM_SHARED((NP, H), jnp.float32),
        pltpu.SemaphoreType.DMA,
        pltpu.SemaphoreType.DMA,
        pltpu.SemaphoreType.DMA,
    ]

    def body(h_hbm, src_hbm, dst_hbm, z_hbm, out_hbm,
             srcg0, srcg1, dst_v, rows0, rows1, agg_sh, sem0, sem1, semg):
        c = lax.axis_index("c")
        s = lax.axis_index("s")
        wid = s * NC + c
        crow = pl.multiple_of(wid * CW, 8)

        row0 = pl.multiple_of(s * RPW, 8)
        pltpu.sync_copy(z_hbm, agg_sh.at[pl.ds(row0, RPW)])
        pltpu.sync_copy(dst_hbm.at[pl.ds(crow, CW)], dst_v)
        pltpu.sync_copy(src_hbm.at[pl.ds(crow, 8)], srcg0)
        pltpu.async_copy(src_hbm.at[pl.ds(crow + 8, 8)], srcg1, semg)
        plsc.subcore_barrier()

        bufs = ((rows0, sem0), (rows1, sem1))
        srcgs = (srcg0, srcg1)
        for b, (rows, sem) in enumerate(bufs):
            pltpu.async_copy(h_hbm.at[srcg0.at[b]], rows, sem)

        def g_step(g, carry):
            for j in range(8):
                i = g * 8 + j
                rows, sem = bufs[j % 2]
                pltpu.make_async_copy(h_hbm.at[srcg0.at[j]], rows, sem).wait()
                pltpu.sync_copy(rows, agg_sh.at[dst_v.at[i]], add=True)

                if j < 6:
                    for b in range(2):
                        @pl.when(lax.rem(g, 2) == b)
                        def _():
                            pltpu.async_copy(h_hbm.at[srcgs[b].at[j + 2]],
                                             rows, sem)
                else:
                    if j == 6:
                        @pl.when(g + 1 < GW)
                        def _():
                            pltpu.make_async_copy(
                                src_hbm.at[pl.ds(crow, 8)], srcg1, semg).wait()

                    @pl.when(g + 1 < GW)
                    def _():
                        for b in range(2):
                            @pl.when(lax.rem(g + 1, 2) == b)
                            def _():
                                pltpu.async_copy(h_hbm.at[srcgs[b].at[j - 6]],
                                                 rows, sem)

            @pl.when(g + 2 < GW)
            def _():
                grow = pl.multiple_of(crow + (g + 2) * 8, 8)
                for b in range(2):
                    @pl.when(lax.rem(g, 2) == b)
                    def _():
                        pltpu.async_copy(src_hbm.at[pl.ds(grow, 8)],
                                         srcgs[b], semg)
            return carry

        lax.fori_loop(0, GW, g_step, 0)
        plsc.subcore_barrier()

        pltpu.sync_copy(agg_sh.at[pl.ds(row0, RPW)],
                        out_hbm.at[pl.ds(c * NP + row0, RPW)])

    return pl.kernel(body, out_type=out_type, mesh=mesh,
                     scratch_types=scratch)


def _make_sc_deg(mesh):
    out_type = jax.ShapeDtypeStruct((NC * NP, DEGW), jnp.float32)
    scratch = [
        pltpu.VMEM((CW, KC), jnp.int32),
        pltpu.VMEM((KC, DEGW), jnp.float32),
        pltpu.VMEM_SHARED((NP, DEGW), jnp.float32),
        pltpu.SemaphoreType.DMA,
    ]

    def body(dst_hbm, z_hbm, ones_hbm,
             out_hbm, dst_v, ones_v, deg_sh, sem):
        c = lax.axis_index("c")
        s = lax.axis_index("s")
        wid = s * NC + c

        row0 = pl.multiple_of(s * RPW, 8)
        pltpu.sync_copy(z_hbm, deg_sh.at[pl.ds(row0, RPW)])
        pltpu.sync_copy(dst_hbm.at[pl.ds(wid * CW, CW)], dst_v)
        pltpu.sync_copy(ones_hbm, ones_v)
        plsc.subcore_barrier()

        for j in range(DGRP):
            pltpu.async_copy(ones_v, deg_sh.at[dst_v.at[j]], sem, add=True)

        def step(i, carry):
            pltpu.make_async_copy(ones_v, deg_sh.at[dst_v.at[0]], sem).wait()
            pltpu.async_copy(ones_v, deg_sh.at[dst_v.at[i + DGRP]],
                             sem, add=True)
            return carry

        lax.fori_loop(0, CW - DGRP, step, 0)
        for j in range(DGRP):
            pltpu.make_async_copy(ones_v, deg_sh.at[dst_v.at[0]], sem).wait()
        plsc.subcore_barrier()

        pltpu.sync_copy(deg_sh.at[pl.ds(row0, RPW)],
                        out_hbm.at[pl.ds(c * NP + row0, RPW)])

    return pl.kernel(body, out_type=out_type, mesh=mesh,
                     scratch_types=scratch)


@functools.cache
def _get_sc_kernels():
    mesh = plsc.VectorSubcoreMesh(core_axis_name="c", subcore_axis_name="s",
                                  num_cores=NC, num_subcores=NS)
    return _make_sc_agg(mesh), _make_sc_deg(mesh)


_R = 1000


def _dense_body(h_ref, p_ref, deg_ref, ws_ref, wn_ref, b_ref, o_ref):
    p = p_ref[...]
    agg = p[0] + p[1]
    d = deg_ref[...]
    deg = jnp.maximum((d[0] + d[1])[:, 0:1], 1.0)
    hn = agg / deg
    acc = jnp.dot(h_ref[...], ws_ref[...], preferred_element_type=jnp.float32)
    acc = acc + jnp.dot(hn, wn_ref[...], preferred_element_type=jnp.float32)
    o_ref[...] = jnp.maximum(acc + b_ref[...], 0.0)


_dense = pl.pallas_call(
    _dense_body,
    grid=(N // _R,),
    in_specs=[
        pl.BlockSpec((_R, H), lambda i: (i, 0)),
        pl.BlockSpec((NC, _R, H), lambda i: (0, i, 0)),
        pl.BlockSpec((NC, _R, DEGW), lambda i: (0, i, 0)),
        pl.BlockSpec((H, H), lambda i: (0, 0)),
        pl.BlockSpec((H, H), lambda i: (0, 0)),
        pl.BlockSpec((1, H), lambda i: (0, 0)),
    ],
    out_specs=pl.BlockSpec((_R, H), lambda i: (i, 0)),
    out_shape=jax.ShapeDtypeStruct((N, H), jnp.float32),
)


def _head_body(h_ref, p_ref, deg_ref, ws_ref, wn_ref, b_ref, g_ref,
               pht_ref, pgt_ref, o_ref):
    p = p_ref[...]
    agg = p[0] + p[1]
    d = deg_ref[...]
    deg = jnp.maximum((d[0] + d[1])[:, 0:1], 1.0)
    hn = agg / deg
    acc = jnp.dot(h_ref[...], ws_ref[...], preferred_element_type=jnp.float32)
    acc = acc + jnp.dot(hn, wn_ref[...], preferred_element_type=jnp.float32)
    h2 = jnp.maximum(acc + b_ref[...], 0.0)
    g = g_ref[...]
    pht = pht_ref[...]
    pgt = pgt_ref[...]
    psq = jnp.sum(pht * pht, axis=0) + jnp.sum(pgt * pgt, axis=0)
    sq = (jnp.sum(h2 * h2, axis=1, keepdims=True)
          + jnp.sum(g * g, axis=1, keepdims=True))
    dotp = (jnp.dot(h2, pht, preferred_element_type=jnp.float32)
            + jnp.dot(g, pgt, preferred_element_type=jnp.float32))
    o_ref[...] = 2.0 * dotp - sq - psq[None, :]


_head = pl.pallas_call(
    _head_body,
    grid=(N // _R,),
    in_specs=[
        pl.BlockSpec((_R, H), lambda i: (i, 0)),
        pl.BlockSpec((NC, _R, H), lambda i: (0, i, 0)),
        pl.BlockSpec((NC, _R, DEGW), lambda i: (0, i, 0)),
        pl.BlockSpec((H, H), lambda i: (0, 0)),
        pl.BlockSpec((H, H), lambda i: (0, 0)),
        pl.BlockSpec((1, H), lambda i: (0, 0)),
        pl.BlockSpec((_R, H), lambda i: (i, 0)),
        pl.BlockSpec((H, C), lambda i: (0, 0)),
        pl.BlockSpec((H, C), lambda i: (0, 0)),
    ],
    out_specs=pl.BlockSpec((_R, C), lambda i: (i, 0)),
    out_shape=jax.ShapeDtypeStruct((N, C), jnp.float32),
)


def kernel(x, gfeat, edge_index, W_self0, W_neigh0, b0,
           W_self1, W_neigh1, b1, proto):
    src = edge_index[0].reshape(EC, KC)
    dst = edge_index[1].reshape(EC, KC)
    z_h = jnp.zeros((RPW, H), jnp.float32)
    z_d = jnp.zeros((RPW, DEGW), jnp.float32)
    ones = jnp.ones((KC, DEGW), jnp.float32)
    b0_2 = b0.reshape(1, H)
    b1_2 = b1.reshape(1, H)
    pht = proto[:, :H].T
    pgt = proto[:, H:].T

    _sc_agg, _sc_deg = _get_sc_kernels()
    degp = _sc_deg(dst, z_d, ones).reshape(NC, NP, DEGW)
    p0 = _sc_agg(x, src, dst, z_h).reshape(NC, NP, H)
    h1 = _dense(x, p0, degp, W_self0, W_neigh0, b0_2)
    p1 = _sc_agg(h1, src, dst, z_h).reshape(NC, NP, H)
    logits = _head(h1, p1, degp, W_self1, W_neigh1, b1_2, gfeat, pht, pgt)
    return logits


# speedup vs baseline: 11.1244x; 1.0516x over previous
import functools

import jax
import jax.numpy as jnp
from jax import lax
from jax.experimental import pallas as pl
from jax.experimental.pallas import tpu as pltpu
from jax.experimental.pallas import tpu_sc as plsc

N = 10000
E = 320000
H = 128
C = 16

NC, NS = 2, 16
NW = NC * NS
EW = E // NW
KC = 50
EC = E // KC
CW = EC // NW
GW = CW // 8
KCD = 125
ECD = E // KCD
CWD = ECD // NW
NP = 10240
RPW = NP // NS
DEGW = 128
DGRP = 8


def _make_sc_agg(mesh):
    out_type = jax.ShapeDtypeStruct((NC * NP, H), jnp.float32)
    scratch = [
        pltpu.VMEM((8, KC), jnp.int32),
        pltpu.VMEM((8, KC), jnp.int32),
        pltpu.VMEM((8, KC), jnp.int32),
        pltpu.VMEM((8, KC), jnp.int32),
        pltpu.VMEM((KC, H), jnp.float32),
        pltpu.VMEM((KC, H), jnp.float32),
        pltpu.VMEM((KC, H), jnp.float32),
        pltpu.VMEM((KC, H), jnp.float32),
        pltpu.VMEM_SHARED((NP, H), jnp.float32),
        pltpu.SemaphoreType.DMA,
        pltpu.SemaphoreType.DMA,
        pltpu.SemaphoreType.DMA,
        pltpu.SemaphoreType.DMA,
        pltpu.SemaphoreType.DMA,
    ]

    def body(h_hbm, src_hbm, dst_hbm, z_hbm, out_hbm,
             srcg0, srcg1, dstg0, dstg1, rows0, rows1, rows2, rows3,
             agg_sh, sem0, sem1, sem2, sem3, semg):
        c = lax.axis_index("c")
        s = lax.axis_index("s")
        wid = s * NC + c
        crow = pl.multiple_of(wid * CW, 8)

        row0 = pl.multiple_of(s * RPW, 8)
        pltpu.sync_copy(z_hbm, agg_sh.at[pl.ds(row0, RPW)])
        pltpu.sync_copy(src_hbm.at[pl.ds(crow, 8)], srcg0)
        pltpu.sync_copy(dst_hbm.at[pl.ds(crow, 8)], dstg0)
        pltpu.async_copy(src_hbm.at[pl.ds(crow + 8, 8)], srcg1, semg)
        pltpu.async_copy(dst_hbm.at[pl.ds(crow + 8, 8)], dstg1, semg)
        plsc.subcore_barrier()

        bufs = ((rows0, sem0), (rows1, sem1), (rows2, sem2), (rows3, sem3))
        srcgs = (srcg0, srcg1)
        dstgs = (dstg0, dstg1)
        for b, (rows, sem) in enumerate(bufs):
            pltpu.async_copy(h_hbm.at[srcg0.at[b]], rows, sem)

        def g_step(g, carry):
            for j in range(8):
                i = g * 8 + j
                rows, sem = bufs[j % 4]
                pltpu.make_async_copy(h_hbm.at[srcg0.at[j % 8]], rows,
                                      sem).wait()
                for b in range(2):
                    @pl.when(lax.rem(g, 2) == b)
                    def _():
                        pltpu.sync_copy(rows, agg_sh.at[dstgs[b].at[j]],
                                        add=True)

                if j < 4:
                    for b in range(2):
                        @pl.when(lax.rem(g, 2) == b)
                        def _():
                            pltpu.async_copy(h_hbm.at[srcgs[b].at[j + 4]],
                                             rows, sem)
                else:
                    if j == 4:
                        @pl.when(g + 1 < GW)
                        def _():
                            pltpu.make_async_copy(
                                src_hbm.at[pl.ds(crow, 8)], srcg1, semg).wait()
                            pltpu.make_async_copy(
                                dst_hbm.at[pl.ds(crow, 8)], dstg1, semg).wait()

                    @pl.when(g + 1 < GW)
                    def _():
                        for b in range(2):
                            @pl.when(lax.rem(g + 1, 2) == b)
                            def _():
                                pltpu.async_copy(h_hbm.at[srcgs[b].at[j - 4]],
                                                 rows, sem)

            @pl.when(g + 2 < GW)
            def _():
                grow = pl.multiple_of(crow + (g + 2) * 8, 8)
                for b in range(2):
                    @pl.when(lax.rem(g, 2) == b)
                    def _():
                        pltpu.async_copy(src_hbm.at[pl.ds(grow, 8)],
                                         srcgs[b], semg)
                        pltpu.async_copy(dst_hbm.at[pl.ds(grow, 8)],
                                         dstgs[b], semg)
            return carry

        lax.fori_loop(0, GW, g_step, 0)
        plsc.subcore_barrier()

        pltpu.sync_copy(agg_sh.at[pl.ds(row0, RPW)],
                        out_hbm.at[pl.ds(c * NP + row0, RPW)])

    return pl.kernel(body, out_type=out_type, mesh=mesh,
                     scratch_types=scratch)


def _make_sc_deg(mesh):
    out_type = jax.ShapeDtypeStruct((NC * NP, DEGW), jnp.float32)
    scratch = [
        pltpu.VMEM((CWD, KCD), jnp.int32),
        pltpu.VMEM((KCD, DEGW), jnp.float32),
        pltpu.VMEM_SHARED((NP, DEGW), jnp.float32),
        pltpu.SemaphoreType.DMA,
    ]

    def body(dst_hbm, z_hbm, ones_hbm,
             out_hbm, dst_v, ones_v, deg_sh, sem):
        c = lax.axis_index("c")
        s = lax.axis_index("s")
        wid = s * NC + c

        row0 = pl.multiple_of(s * RPW, 8)
        pltpu.sync_copy(z_hbm, deg_sh.at[pl.ds(row0, RPW)])
        pltpu.sync_copy(dst_hbm.at[pl.ds(wid * CWD, CWD)], dst_v)
        pltpu.sync_copy(ones_hbm, ones_v)
        plsc.subcore_barrier()

        for j in range(DGRP):
            pltpu.async_copy(ones_v, deg_sh.at[dst_v.at[j]], sem, add=True)

        def step(i, carry):
            pltpu.make_async_copy(ones_v, deg_sh.at[dst_v.at[0]], sem).wait()
            pltpu.async_copy(ones_v, deg_sh.at[dst_v.at[i + DGRP]],
                             sem, add=True)
            return carry

        lax.fori_loop(0, CWD - DGRP, step, 0)
        for j in range(DGRP):
            pltpu.make_async_copy(ones_v, deg_sh.at[dst_v.at[0]], sem).wait()
        plsc.subcore_barrier()

        pltpu.sync_copy(deg_sh.at[pl.ds(row0, RPW)],
                        out_hbm.at[pl.ds(c * NP + row0, RPW)])

    return pl.kernel(body, out_type=out_type, mesh=mesh,
                     scratch_types=scratch)


@functools.cache
def _get_sc_kernels():
    mesh = plsc.VectorSubcoreMesh(core_axis_name="c", subcore_axis_name="s",
                                  num_cores=NC, num_subcores=NS)
    return _make_sc_agg(mesh), _make_sc_deg(mesh)


_R = 1000


def _dense_body(h_ref, p_ref, deg_ref, ws_ref, wn_ref, b_ref, o_ref):
    p = p_ref[...]
    agg = p[0] + p[1]
    d = deg_ref[...]
    deg = jnp.maximum((d[0] + d[1])[:, 0:1], 1.0)
    hn = agg / deg
    acc = jnp.dot(h_ref[...], ws_ref[...], preferred_element_type=jnp.float32)
    acc = acc + jnp.dot(hn, wn_ref[...], preferred_element_type=jnp.float32)
    o_ref[...] = jnp.maximum(acc + b_ref[...], 0.0)


_dense = pl.pallas_call(
    _dense_body,
    grid=(N // _R,),
    in_specs=[
        pl.BlockSpec((_R, H), lambda i: (i, 0)),
        pl.BlockSpec((NC, _R, H), lambda i: (0, i, 0)),
        pl.BlockSpec((NC, _R, DEGW), lambda i: (0, i, 0)),
        pl.BlockSpec((H, H), lambda i: (0, 0)),
        pl.BlockSpec((H, H), lambda i: (0, 0)),
        pl.BlockSpec((1, H), lambda i: (0, 0)),
    ],
    out_specs=pl.BlockSpec((_R, H), lambda i: (i, 0)),
    out_shape=jax.ShapeDtypeStruct((N, H), jnp.float32),
)


def _head_body(h_ref, p_ref, deg_ref, ws_ref, wn_ref, b_ref, g_ref,
               pht_ref, pgt_ref, o_ref):
    p = p_ref[...]
    agg = p[0] + p[1]
    d = deg_ref[...]
    deg = jnp.maximum((d[0] + d[1])[:, 0:1], 1.0)
    hn = agg / deg
    acc = jnp.dot(h_ref[...], ws_ref[...], preferred_element_type=jnp.float32)
    acc = acc + jnp.dot(hn, wn_ref[...], preferred_element_type=jnp.float32)
    h2 = jnp.maximum(acc + b_ref[...], 0.0)
    g = g_ref[...]
    pht = pht_ref[...]
    pgt = pgt_ref[...]
    psq = jnp.sum(pht * pht, axis=0) + jnp.sum(pgt * pgt, axis=0)
    sq = (jnp.sum(h2 * h2, axis=1, keepdims=True)
          + jnp.sum(g * g, axis=1, keepdims=True))
    dotp = (jnp.dot(h2, pht, preferred_element_type=jnp.float32)
            + jnp.dot(g, pgt, preferred_element_type=jnp.float32))
    o_ref[...] = 2.0 * dotp - sq - psq[None, :]


_head = pl.pallas_call(
    _head_body,
    grid=(N // _R,),
    in_specs=[
        pl.BlockSpec((_R, H), lambda i: (i, 0)),
        pl.BlockSpec((NC, _R, H), lambda i: (0, i, 0)),
        pl.BlockSpec((NC, _R, DEGW), lambda i: (0, i, 0)),
        pl.BlockSpec((H, H), lambda i: (0, 0)),
        pl.BlockSpec((H, H), lambda i: (0, 0)),
        pl.BlockSpec((1, H), lambda i: (0, 0)),
        pl.BlockSpec((_R, H), lambda i: (i, 0)),
        pl.BlockSpec((H, C), lambda i: (0, 0)),
        pl.BlockSpec((H, C), lambda i: (0, 0)),
    ],
    out_specs=pl.BlockSpec((_R, C), lambda i: (i, 0)),
    out_shape=jax.ShapeDtypeStruct((N, C), jnp.float32),
)


def kernel(x, gfeat, edge_index, W_self0, W_neigh0, b0,
           W_self1, W_neigh1, b1, proto):
    src = edge_index[0].reshape(EC, KC)
    dst = edge_index[1].reshape(EC, KC)
    dstd = edge_index[1].reshape(ECD, KCD)
    z_h = jnp.zeros((RPW, H), jnp.float32)
    z_d = jnp.zeros((RPW, DEGW), jnp.float32)
    ones = jnp.ones((KCD, DEGW), jnp.float32)
    b0_2 = b0.reshape(1, H)
    b1_2 = b1.reshape(1, H)
    pht = proto[:, :H].T
    pgt = proto[:, H:].T

    _sc_agg, _sc_deg = _get_sc_kernels()
    degp = _sc_deg(dstd, z_d, ones).reshape(NC, NP, DEGW)
    p0 = _sc_agg(x, src, dst, z_h).reshape(NC, NP, H)
    h1 = _dense(x, p0, degp, W_self0, W_neigh0, b0_2)
    p1 = _sc_agg(h1, src, dst, z_h).reshape(NC, NP, H)
    logits = _head(h1, p1, degp, W_self1, W_neigh1, b1_2, gfeat, pht, pgt)
    return logits
